# baseline (speedup 1.0000x reference)
import functools

import numpy as np
import jax
import jax.numpy as jnp
from jax import lax
from jax.experimental import pallas as pl
from jax.experimental.pallas import tpu as pltpu
from jax.experimental.pallas import tpu_sc as plsc

B = 1024
L = 50
D = 64
V = 100000
NUM_NEG = 2
S = NUM_NEG * L
SP = 104
SEQP = 56

_KS = (0, 42, 0 ^ 42 ^ 0x1BD11BDA)
_ROT = ((13, 15, 26, 6), (17, 29, 16, 24))
_MININT = -(2 ** 31)


def _rotl(x, d):
    return lax.shift_left(x, np.int32(d)) | lax.shift_right_logical(
        x, np.int32(32 - d))


def _i32(c):
    c = c & 0xFFFFFFFF
    if c >= 2 ** 31:
        c -= 2 ** 32
    return np.int32(c)


def _threefry_bits(x0, x1):
    if _KS[0]:
        x0 = x0 + _i32(_KS[0])
    x1 = x1 + _i32(_KS[1])
    for r in range(5):
        for d in _ROT[r % 2]:
            x0 = x0 + x1
            x1 = _rotl(x1, d)
            x1 = x1 ^ x0
        inj0 = _KS[(r + 1) % 3]
        inj1 = _KS[(r + 2) % 3] + (r + 1)
        if inj0:
            x0 = x0 + _i32(inj0)
        if inj1:
            x1 = x1 + _i32(inj1)
    return x0 ^ x1


def _make_sampler(BB, VV, C):
    n_chunks = -(-VV // C)

    base = ((np.arange(SP, dtype=np.int64)[:, None] * BB
             + np.arange(BB, dtype=np.int64)[None, :]) * VV).T
    base_hi = (base >> 32).astype(np.int32)[:, :, None]
    base_lo = (base & 0xFFFFFFFF).astype(np.uint32).view(np.int32)[:, :, None]

    def body(seq_ref, bhi_ref, blo_ref, out_ref):
        bhi = bhi_ref[0]
        blo = blo_ref[0]
        seq = seq_ref[0]

        def chunk(j, carry):
            best, bidx = carry
            iota = (lax.broadcasted_iota(jnp.int32, (1, C), 1)
                    + j * np.int32(C))
            hit = jnp.any(seq == iota, axis=0, keepdims=True)
            allowed = (~hit) & (iota != 0) & (iota < np.int32(VV))
            n_lo = blo + iota
            cbit = ((n_lo ^ np.int32(_MININT))
                    < (iota ^ np.int32(_MININT))).astype(jnp.int32)
            n_hi = bhi + cbit
            bits = _threefry_bits(n_hi, n_lo)
            key = lax.shift_right_logical(bits, np.int32(9))
            key = jnp.where(allowed, key, np.int32(-1))
            cmax = jnp.max(key, axis=1, keepdims=True)
            cidx = jnp.min(jnp.where(key == cmax, iota, np.int32(2 ** 30)),
                           axis=1, keepdims=True)
            upd = cmax > best
            return jnp.where(upd, cmax, best), jnp.where(upd, cidx, bidx)

        best0 = jnp.full((SP, 1), -1, jnp.int32)
        bidx0 = jnp.zeros((SP, 1), jnp.int32)
        _, bidx = lax.fori_loop(0, n_chunks, chunk, (best0, bidx0))
        out_ref[...] = bidx[None]

    call = pl.pallas_call(
        body,
        grid=(BB,),
        in_specs=[
            pl.BlockSpec((1, SEQP, 1), lambda b: (b, 0, 0)),
            pl.BlockSpec((1, SP, 1), lambda b: (b, 0, 0)),
            pl.BlockSpec((1, SP, 1), lambda b: (b, 0, 0)),
        ],
        out_specs=pl.BlockSpec((1, SP, 1), lambda b: (b, 0, 0)),
        out_shape=jax.ShapeDtypeStruct((BB, SP, 1), jnp.int32),
    )

    def run(seq3):
        return call(seq3, jnp.asarray(base_hi), jnp.asarray(base_lo))

    return run


_sampler = _make_sampler(B, V, 512)


def _make_gather(N, Dd):
    info = plsc.get_sparse_core_info()
    nw = info.num_cores * info.num_subcores
    per_w = N // nw
    CH = 120
    n_ch = per_w // CH
    assert per_w % CH == 0 and N % nw == 0
    mesh = plsc.VectorSubcoreMesh(core_axis_name="c", subcore_axis_name="s")

    @functools.partial(
        pl.kernel, mesh=mesh,
        out_type=jax.ShapeDtypeStruct((N, Dd), jnp.float32),
        compiler_params=pltpu.CompilerParams(use_tc_tiling_on_sc=False),
        scratch_types=[
            pltpu.VMEM((CH,), jnp.int32),
            pltpu.VMEM((CH, Dd), jnp.float32),
            pltpu.SemaphoreType.DMA,
        ],
    )
    def gk(emb_hbm, idx_hbm, out_hbm, idx_v, rows_v, sem):
        wid = lax.axis_index("s") * info.num_cores + lax.axis_index("c")
        base = wid * np.int32(per_w)

        def step(c, tok):
            off = pl.multiple_of(base + c * np.int32(CH), 8)
            pltpu.sync_copy(idx_hbm.at[pl.ds(off, CH)], idx_v)
            pltpu.async_copy(emb_hbm.at[idx_v], rows_v, sem).wait()
            pltpu.sync_copy(rows_v, out_hbm.at[pl.ds(off, CH)])
            return tok

        lax.fori_loop(0, n_ch, step, 0)

    return gk


_gather_cache = {}


def _gather(emb, idx):
    gk = _gather_cache.get("gk")
    if gk is None:
        gk = _make_gather(B * L * (1 + NUM_NEG), D)
        _gather_cache["gk"] = gk
    return gk(emb, idx)


def _make_scores(N, Dd, R):
    def body(q_ref, p_ref, n_ref, t_ref, out_ref):
        q = q_ref[...]
        sp = jnp.sum(q * p_ref[...], axis=1, keepdims=True)
        n2 = n_ref[...]
        s0 = jnp.sum(q * n2[:, :Dd], axis=1, keepdims=True)
        s1 = jnp.sum(q * n2[:, Dd:], axis=1, keepdims=True)
        sp = jnp.where(t_ref[...] == 0, -jnp.inf, sp)
        out_ref[:, 0:1] = sp
        out_ref[:, 1:2] = s0
        out_ref[:, 2:3] = s1

    return pl.pallas_call(
        body,
        grid=(N // R,),
        in_specs=[
            pl.BlockSpec((R, Dd), lambda i: (i, 0)),
            pl.BlockSpec((R, Dd), lambda i: (i, 0)),
            pl.BlockSpec((R, 2 * Dd), lambda i: (i, 0)),
            pl.BlockSpec((R, 1), lambda i: (i, 0)),
        ],
        out_specs=pl.BlockSpec((R, 3), lambda i: (i, 0)),
        out_shape=jax.ShapeDtypeStruct((N, 3), jnp.float32),
    )


_scores = _make_scores(B * L, D, 512)


def kernel(query, item_emb, user_seq, target_item):
    seq3 = jnp.concatenate(
        [user_seq.astype(jnp.int32),
         jnp.zeros((B, SEQP - L), jnp.int32)], axis=1)[:, :, None]
    samples = _sampler(seq3)
    neg = samples[:, :S, 0].reshape(B * L * NUM_NEG)
    tgt = target_item.astype(jnp.int32).reshape(B * L)
    all_idx = jnp.concatenate([tgt, neg])
    rows = _gather(item_emb, all_idx)
    out = _scores(
        query.reshape(B * L, D),
        rows[:B * L],
        rows[B * L:].reshape(B * L, 2 * D),
        tgt.reshape(B * L, 1),
    )
    return out.reshape(B, L, 1 + NUM_NEG)
